# baseline (speedup 1.0000x reference)
import functools

import jax
import jax.numpy as jnp
from jax import lax
from jax.experimental import pallas as pl
from jax.experimental.pallas import tpu as pltpu
from jax.experimental.pallas import tpu_sc as plsc

SEQ_LEN = 8192
DIM = 1024
NUM_CORES = 2
NUM_SUBCORES = 16
NUM_WORKERS = NUM_CORES * NUM_SUBCORES
ROWS_PER_WORKER = SEQ_LEN // NUM_WORKERS
CHUNK = 8
NCHUNKS = ROWS_PER_WORKER // CHUNK
LANES = 16

EBYTES = CHUNK * 2 * DIM * 4
PBYTES = CHUNK * DIM * 4


def _sc_body(emb_hbm, pe_hbm, out_hbm,
             eb0, eb1, pb0, pb1, ob0, ob1,
             sei0, sei1, spi0, spi1, so0, so1):
    wid = lax.axis_index("s") * NUM_CORES + lax.axis_index("c")
    base = wid * ROWS_PER_WORKER
    ebufs, pbufs, obufs = (eb0, eb1), (pb0, pb1), (ob0, ob1)
    sei, spi, so = (sei0, sei1), (spi0, spi1), (so0, so1)

    def start_in(g, s):
        r0 = base + g * CHUNK
        pltpu.async_copy(emb_hbm.at[pl.ds(r0, CHUNK)], ebufs[s], sei[s])
        pltpu.async_copy(pe_hbm.at[pl.ds(r0, CHUNK)], pbufs[s], spi[s])

    def wait_in(s):
        pltpu.make_async_copy(emb_hbm.at[pl.ds(0, CHUNK)], ebufs[s], sei[s]).wait()
        pltpu.make_async_copy(pe_hbm.at[pl.ds(0, CHUNK)], pbufs[s], spi[s]).wait()

    def start_out(g, s):
        r0 = base + g * CHUNK
        pltpu.async_copy(obufs[s], out_hbm.at[pl.ds(r0, CHUNK)], so[s])

    def wait_out(s):
        pltpu.make_async_copy(obufs[s], out_hbm.at[pl.ds(0, CHUNK)], so[s]).wait()

    def compute(s):
        eb, pb, ob = ebufs[s], pbufs[s], obufs[s]

        def row_step(r, c):
            for j in range(DIM // LANES):
                pv = pb[r, pl.ds(j * LANES, LANES)]
                ob[r, pl.ds(j * LANES, LANES)] = (
                    eb[r, pl.ds(j * LANES, LANES)] + pv)
                ob[r, pl.ds(DIM + j * LANES, LANES)] = (
                    eb[r, pl.ds(DIM + j * LANES, LANES)] + pv)
            return c

        lax.fori_loop(0, CHUNK, row_step, 0)

    start_in(0, 0)
    start_in(1, 1)

    for s in range(2):
        wait_in(s)
        compute(s)
        start_out(s, s)
        start_in(2 + s, s)

    def round_body(i, c):
        for s in range(2):
            g = 2 * i + s
            wait_out(s)
            wait_in(s)
            compute(s)
            start_out(g, s)
            start_in(g + 2, s)
        return c

    lax.fori_loop(1, NCHUNKS // 2 - 1, round_body, 0)

    for s in range(2):
        g = NCHUNKS - 2 + s
        wait_out(s)
        wait_in(s)
        compute(s)
        start_out(g, s)
    for s in range(2):
        wait_out(s)


@jax.jit
def kernel(emb, pe_table):
    seq_len, batch, dim = emb.shape
    emb2 = emb.reshape(seq_len, batch * dim)
    sc_kernel = functools.partial(
        pl.kernel,
        out_type=jax.ShapeDtypeStruct((seq_len, batch * dim), emb.dtype),
        mesh=plsc.VectorSubcoreMesh(core_axis_name="c", subcore_axis_name="s"),
        compiler_params=pltpu.CompilerParams(use_tc_tiling_on_sc=True),
        scratch_types=[
            pltpu.VMEM((CHUNK, batch * dim), jnp.float32),
            pltpu.VMEM((CHUNK, batch * dim), jnp.float32),
            pltpu.VMEM((CHUNK, dim), jnp.float32),
            pltpu.VMEM((CHUNK, dim), jnp.float32),
            pltpu.VMEM((CHUNK, batch * dim), jnp.float32),
            pltpu.VMEM((CHUNK, batch * dim), jnp.float32),
            pltpu.SemaphoreType.DMA,
            pltpu.SemaphoreType.DMA,
            pltpu.SemaphoreType.DMA,
            pltpu.SemaphoreType.DMA,
            pltpu.SemaphoreType.DMA,
            pltpu.SemaphoreType.DMA,
        ],
    )(_sc_body)
    out = sc_kernel(emb2, pe_table)
    return out.reshape(seq_len, batch, dim)


# speedup vs baseline: 2.4087x; 2.4087x over previous
import functools

import jax
import jax.numpy as jnp
from jax import lax
from jax.experimental import pallas as pl
from jax.experimental.pallas import tpu as pltpu
from jax.experimental.pallas import tpu_sc as plsc

SEQ_LEN = 8192
BATCH = 2
DIM = 1024
NUM_CORES = 2
NUM_SUBCORES = 16
NUM_WORKERS = NUM_CORES * NUM_SUBCORES
ROWS_PER_WORKER = SEQ_LEN // NUM_WORKERS
CHUNK = 8
NCHUNKS = ROWS_PER_WORKER // CHUNK
LANES = 16


def _sc_body(emb_hbm, pe_hbm, out_hbm,
             eb0, eb1, pb0, pb1, ob0, ob1,
             sei0, sei1, spi0, spi1, so0, so1):
    wid = lax.axis_index("s") * NUM_CORES + lax.axis_index("c")
    base = wid * ROWS_PER_WORKER
    ebufs, pbufs, obufs = (eb0, eb1), (pb0, pb1), (ob0, ob1)
    sei, spi, so = (sei0, sei1), (spi0, spi1), (so0, so1)

    def start_in(g, s):
        r0 = base + g * CHUNK
        pltpu.async_copy(emb_hbm.at[pl.ds(r0, CHUNK)], ebufs[s], sei[s])
        pltpu.async_copy(pe_hbm.at[pl.ds(r0, CHUNK)], pbufs[s], spi[s])

    def wait_in(s):
        pltpu.make_async_copy(emb_hbm.at[pl.ds(0, CHUNK)], ebufs[s], sei[s]).wait()
        pltpu.make_async_copy(pe_hbm.at[pl.ds(0, CHUNK)], pbufs[s], spi[s]).wait()

    def start_out(g, s):
        r0 = base + g * CHUNK
        pltpu.async_copy(obufs[s], out_hbm.at[pl.ds(r0, CHUNK)], so[s])

    def wait_out(s):
        pltpu.make_async_copy(obufs[s], out_hbm.at[pl.ds(0, CHUNK)], so[s]).wait()

    def compute(s):
        eb, pb, ob = ebufs[s], pbufs[s], obufs[s]

        def row_step(r, c):
            for j in range(DIM // LANES):
                pv = pb[r, pl.ds(j * LANES, LANES)]
                ob[r, 0, pl.ds(j * LANES, LANES)] = (
                    eb[r, 0, pl.ds(j * LANES, LANES)] + pv)
                ob[r, 1, pl.ds(j * LANES, LANES)] = (
                    eb[r, 1, pl.ds(j * LANES, LANES)] + pv)
            return c

        lax.fori_loop(0, CHUNK, row_step, 0)

    start_in(0, 0)
    start_in(1, 1)

    for s in range(2):
        wait_in(s)
        compute(s)
        start_out(s, s)
        start_in(2 + s, s)

    def round_body(i, c):
        for s in range(2):
            g = 2 * i + s
            wait_out(s)
            wait_in(s)
            compute(s)
            start_out(g, s)
            start_in(g + 2, s)
        return c

    lax.fori_loop(1, NCHUNKS // 2 - 1, round_body, 0)

    for s in range(2):
        g = NCHUNKS - 2 + s
        wait_out(s)
        wait_in(s)
        compute(s)
        start_out(g, s)
    for s in range(2):
        wait_out(s)


@jax.jit
def kernel(emb, pe_table):
    seq_len, batch, dim = emb.shape
    sc_kernel = functools.partial(
        pl.kernel,
        out_type=jax.ShapeDtypeStruct((seq_len, batch, dim), emb.dtype),
        mesh=plsc.VectorSubcoreMesh(core_axis_name="c", subcore_axis_name="s"),
        scratch_types=[
            pltpu.VMEM((CHUNK, BATCH, DIM), jnp.float32),
            pltpu.VMEM((CHUNK, BATCH, DIM), jnp.float32),
            pltpu.VMEM((CHUNK, DIM), jnp.float32),
            pltpu.VMEM((CHUNK, DIM), jnp.float32),
            pltpu.VMEM((CHUNK, BATCH, DIM), jnp.float32),
            pltpu.VMEM((CHUNK, BATCH, DIM), jnp.float32),
            pltpu.SemaphoreType.DMA,
            pltpu.SemaphoreType.DMA,
            pltpu.SemaphoreType.DMA,
            pltpu.SemaphoreType.DMA,
            pltpu.SemaphoreType.DMA,
            pltpu.SemaphoreType.DMA,
        ],
    )(_sc_body)
    return sc_kernel(emb, pe_table)
